# Initial kernel scaffold; baseline (speedup 1.0000x reference)
import functools

import jax
import jax.numpy as jnp
from jax import lax
from jax.experimental import pallas as pl
from jax.experimental.pallas import tpu as pltpu
from jax.experimental.pallas import tpu_sc as plsc


def _build(N, V, D, T):
    info = plsc.get_sparse_core_info()
    NC, NS, L = info.num_cores, info.num_subcores, info.num_lanes
    NW = NC * NS
    C = 128
    assert N % (NW * C) == 0 and D % L == 0
    n_chunks = (N // NW) // C
    nd = D // L

    mesh = plsc.VectorSubcoreMesh(core_axis_name="c", subcore_axis_name="s")

    @functools.partial(
        pl.kernel,
        mesh=mesh,
        out_type=jax.ShapeDtypeStruct((N, D), jnp.float32),
        scratch_types=[
            pltpu.VMEM((C,), jnp.int32),
            pltpu.VMEM((C,), jnp.int32),
            pltpu.VMEM((C, D), jnp.float32),
            pltpu.VMEM((T, D), jnp.float32),
            pltpu.SemaphoreType.DMA,
        ],
    )
    def k(ids_hbm, tids_hbm, table_hbm, tt_hbm, out_hbm, idx_v, tid_v, rows_v, tt_v, gsem):
        wid = lax.axis_index("s") * NC + lax.axis_index("c")
        pltpu.sync_copy(tt_hbm, tt_v)
        row0 = [tt_v[0, pl.ds(d * L, L)] for d in range(nd)]
        row1 = [tt_v[1, pl.ds(d * L, L)] for d in range(nd)]

        @pl.loop(0, n_chunks)
        def _chunk(g):
            chunk_row = wid * n_chunks + g
            base = chunk_row * C
            pltpu.sync_copy(ids_hbm.at[chunk_row], idx_v)
            pltpu.sync_copy(tids_hbm.at[pl.ds(base, C)], tid_v)
            pltpu.async_copy(table_hbm.at[idx_v], rows_v, gsem).wait()

            @pl.loop(0, C)
            def _tok(i):
                t = jnp.full((L,), tid_v[i], dtype=jnp.int32) > 0
                for d in range(nd):
                    sel = jnp.where(t, row1[d], row0[d])
                    rows_v[i, pl.ds(d * L, L)] += sel

            pltpu.sync_copy(rows_v, out_hbm.at[pl.ds(base, C)])

    return k, C


def kernel(input_ids, token_type_ids, word_embeddings, token_type_embeddings):
    B, S = input_ids.shape
    V, D = word_embeddings.shape
    T = token_type_embeddings.shape[0]
    N = B * S
    k, C = _build(N, V, D, T)
    ids = input_ids.reshape(N // C, C).astype(jnp.int32)
    tids = token_type_ids.reshape(N).astype(jnp.int32)
    out = k(ids, tids, word_embeddings, token_type_embeddings)
    return out.reshape(B, S, D)


# speedup vs baseline: 7.7754x; 7.7754x over previous
import functools

import jax
import jax.numpy as jnp
from jax import lax
from jax.experimental import pallas as pl
from jax.experimental.pallas import tpu as pltpu
from jax.experimental.pallas import tpu_sc as plsc


def _build(N, V, D, T):
    info = plsc.get_sparse_core_info()
    NC, NS, L = info.num_cores, info.num_subcores, info.num_lanes
    NW = NC * NS
    C = 128
    assert N % (NW * C) == 0 and D % L == 0
    n_chunks = (N // NW) // C
    nd = D // L

    mesh = plsc.VectorSubcoreMesh(core_axis_name="c", subcore_axis_name="s")

    @functools.partial(
        pl.kernel,
        mesh=mesh,
        out_type=jax.ShapeDtypeStruct((N, D), jnp.float32),
        scratch_types=[
            pltpu.VMEM((C,), jnp.int32),
            pltpu.VMEM((C,), jnp.int32),
            pltpu.VMEM((C, D), jnp.float32),
            pltpu.VMEM((T, D), jnp.float32),
            pltpu.SemaphoreType.DMA,
        ],
    )
    def k(ids_hbm, tids_hbm, table_hbm, tt_hbm, out_hbm, idx_v, tid_v, rows_v, tt_v, gsem):
        wid = lax.axis_index("s") * NC + lax.axis_index("c")
        pltpu.sync_copy(tt_hbm, tt_v)
        row0 = [tt_v[0, pl.ds(d * L, L)] for d in range(nd)]
        diff = [tt_v[1, pl.ds(d * L, L)] - tt_v[0, pl.ds(d * L, L)] for d in range(nd)]

        @pl.loop(0, n_chunks)
        def _chunk(g):
            chunk_row = wid * n_chunks + g
            base = chunk_row * C
            pltpu.sync_copy(ids_hbm.at[chunk_row], idx_v)
            pltpu.sync_copy(tids_hbm.at[pl.ds(base, C)], tid_v)
            pltpu.async_copy(table_hbm.at[idx_v], rows_v, gsem).wait()

            @pl.loop(0, C // L)
            def _grp(j):
                tv = tid_v[pl.ds(j * L, L)].astype(jnp.float32)
                for kk in range(L):
                    tf = jnp.full((L,), tv[kk], dtype=jnp.float32)
                    i = j * L + kk
                    for d in range(nd):
                        rows_v[i, pl.ds(d * L, L)] += row0[d] + tf * diff[d]

            pltpu.sync_copy(rows_v, out_hbm.at[pl.ds(base, C)])

    return k, C


def kernel(input_ids, token_type_ids, word_embeddings, token_type_embeddings):
    B, S = input_ids.shape
    V, D = word_embeddings.shape
    T = token_type_embeddings.shape[0]
    N = B * S
    k, C = _build(N, V, D, T)
    ids = input_ids.reshape(N // C, C).astype(jnp.int32)
    tids = token_type_ids.reshape(N).astype(jnp.int32)
    out = k(ids, tids, word_embeddings, token_type_embeddings)
    return out.reshape(B, S, D)


# speedup vs baseline: 17.7749x; 2.2860x over previous
import functools

import jax
import jax.numpy as jnp
from jax import lax
from jax.experimental import pallas as pl
from jax.experimental.pallas import tpu as pltpu
from jax.experimental.pallas import tpu_sc as plsc


def _build(N, V, D, T):
    info = plsc.get_sparse_core_info()
    NC, NS, L = info.num_cores, info.num_subcores, info.num_lanes
    NW = NC * NS
    G = 128
    C = 256
    nj = C // G
    assert N % (NW * C) == 0 and D % L == 0 and C % G == 0
    n_chunks = (N // NW) // C
    assert n_chunks % 2 == 0
    nd = D // L

    mesh = plsc.VectorSubcoreMesh(core_axis_name="c", subcore_axis_name="s")

    @functools.partial(
        pl.kernel,
        mesh=mesh,
        out_type=jax.ShapeDtypeStruct((N, D), jnp.float32),
        scratch_types=[
            pltpu.VMEM((2, 2 * nj, G), jnp.int32),
            pltpu.VMEM((2, C, D), jnp.float32),
            pltpu.VMEM((T, D), jnp.float32),
            pltpu.SemaphoreType.DMA,
            pltpu.SemaphoreType.DMA,
            pltpu.SemaphoreType.DMA,
            pltpu.SemaphoreType.DMA,
            pltpu.SemaphoreType.DMA,
            pltpu.SemaphoreType.DMA,
        ],
    )
    def k(it_hbm, table_hbm, tt_hbm, out_hbm, it_v, rows_v, tt_v,
          is0, is1, gs0, gs1, ss0, ss1):
        wid = lax.axis_index("s") * NC + lax.axis_index("c")
        isems, gsems, ssems = [is0, is1], [gs0, gs1], [ss0, ss1]
        pltpu.sync_copy(tt_hbm, tt_v)

        def it_copy(g, b):
            pltpu.async_copy(it_hbm.at[wid * n_chunks + g], it_v.at[b], isems[b])

        def wait_it(b):
            pltpu.make_async_copy(it_hbm.at[0], it_v.at[b], isems[b]).wait()

        def gathers(b):
            for j in range(nj):
                pltpu.async_copy(table_hbm.at[it_v.at[b, j]],
                                 rows_v.at[b, pl.ds(j * G, G)], gsems[b])

        def wait_gathers(b):
            for j in range(nj):
                pltpu.make_async_copy(table_hbm.at[it_v.at[b, j]],
                                      rows_v.at[b, pl.ds(j * G, G)], gsems[b]).wait()

        def scatter(g, b):
            base = (wid * n_chunks + g) * C
            pltpu.async_copy(rows_v.at[b], out_hbm.at[pl.ds(base, C)], ssems[b])

        def wait_scatter(b):
            pltpu.make_async_copy(rows_v.at[b], out_hbm.at[pl.ds(0, C)], ssems[b]).wait()

        def compute(b):
            @pl.loop(0, C // L)
            def _grp(j):
                row0 = [tt_v[0, pl.ds(d * L, L)] for d in range(nd)]
                diff = [tt_v[1, pl.ds(d * L, L)] - row0[d] for d in range(nd)]
                tv = it_v[b, nj + j // (G // L), pl.ds((j % (G // L)) * L, L)]
                tvf = tv.astype(jnp.float32)
                for kk in range(L):
                    tf = jnp.full((L,), tvf[kk], dtype=jnp.float32)
                    i = j * L + kk
                    for d in range(nd):
                        rows_v[b, i, pl.ds(d * L, L)] += row0[d] + tf * diff[d]

        it_copy(0, 0)
        it_copy(1, 1)
        wait_it(0)
        gathers(0)

        @pl.loop(0, n_chunks // 2)
        def _outer(go):
            for b in range(2):
                g = go * 2 + b
                nb = 1 - b
                if b == 0:
                    @pl.when(go > 0)
                    def _():
                        wait_scatter(nb)
                    wait_it(nb)
                    gathers(nb)
                else:
                    wait_scatter(nb)

                    @pl.when(go < n_chunks // 2 - 1)
                    def _():
                        wait_it(nb)
                        gathers(nb)
                wait_gathers(b)
                compute(b)

                @pl.when(go < n_chunks // 2 - 1)
                def _():
                    it_copy(g + 2, b)
                scatter(g, b)

        wait_scatter(1)

    return k, C, G, NW


def kernel(input_ids, token_type_ids, word_embeddings, token_type_embeddings):
    B, S = input_ids.shape
    V, D = word_embeddings.shape
    T = token_type_embeddings.shape[0]
    N = B * S
    k, C, G, NW = _build(N, V, D, T)
    nj = C // G
    ids = input_ids.reshape(N // C, nj, G).astype(jnp.int32)
    tids = token_type_ids.reshape(N // C, nj, G).astype(jnp.int32)
    it = jnp.concatenate([ids, tids], axis=1)
    out = k(it, word_embeddings, token_type_embeddings)
    return out.reshape(B, S, D)


# speedup vs baseline: 17.9514x; 1.0099x over previous
import functools

import jax
import jax.numpy as jnp
from jax import lax
from jax.experimental import pallas as pl
from jax.experimental.pallas import tpu as pltpu
from jax.experimental.pallas import tpu_sc as plsc

_R = 4


def _build(N, V, D, T):
    info = plsc.get_sparse_core_info()
    NC, NS, L = info.num_cores, info.num_subcores, info.num_lanes
    NW = NC * NS
    C = 128
    assert N % (NW * C) == 0 and D % L == 0
    n_chunks = (N // NW) // C
    assert n_chunks % _R == 0 and n_chunks >= 2 * _R
    n_outer = n_chunks // _R
    nd = D // L

    mesh = plsc.VectorSubcoreMesh(core_axis_name="c", subcore_axis_name="s")

    @functools.partial(
        pl.kernel,
        mesh=mesh,
        out_type=jax.ShapeDtypeStruct((N, D), jnp.float32),
        scratch_types=[
            pltpu.VMEM((_R, 2, C), jnp.int32),
            pltpu.VMEM((_R, C, D), jnp.float32),
            pltpu.VMEM((T, D), jnp.float32),
        ]
        + [pltpu.SemaphoreType.DMA] * (3 * _R),
    )
    def k(it_hbm, table_hbm, tt_hbm, out_hbm, it_v, rows_v, tt_v, *sems):
        isems, gsems, ssems = sems[:_R], sems[_R:2 * _R], sems[2 * _R:]
        wid = lax.axis_index("s") * NC + lax.axis_index("c")
        pltpu.sync_copy(tt_hbm, tt_v)

        def it_copy(g, b):
            pltpu.async_copy(it_hbm.at[wid * n_chunks + g], it_v.at[b], isems[b])

        def wait_it(b):
            pltpu.make_async_copy(it_hbm.at[0], it_v.at[b], isems[b]).wait()

        def gather(b):
            pltpu.async_copy(table_hbm.at[it_v.at[b, 0]], rows_v.at[b], gsems[b])

        def wait_gather(b):
            pltpu.make_async_copy(table_hbm.at[it_v.at[b, 0]],
                                  rows_v.at[b], gsems[b]).wait()

        def scatter(g, b):
            base = (wid * n_chunks + g) * C
            pltpu.async_copy(rows_v.at[b], out_hbm.at[pl.ds(base, C)], ssems[b])

        def wait_scatter(b):
            pltpu.make_async_copy(rows_v.at[b], out_hbm.at[pl.ds(0, C)], ssems[b]).wait()

        def compute(b):
            @pl.loop(0, C // L)
            def _grp(j):
                row0 = [tt_v[0, pl.ds(d * L, L)] for d in range(nd)]
                diff = [tt_v[1, pl.ds(d * L, L)] - row0[d] for d in range(nd)]
                tvf = it_v[b, 1, pl.ds(j * L, L)].astype(jnp.float32)
                for kk in range(L):
                    tf = jnp.full((L,), tvf[kk], dtype=jnp.float32)
                    i = j * L + kk
                    for d in range(nd):
                        rows_v[b, i, pl.ds(d * L, L)] += row0[d] + tf * diff[d]

        for b in range(_R):
            it_copy(b, b)
        for b in range(2):
            wait_it(b)
            gather(b)

        @pl.loop(0, n_outer)
        def _outer(go):
            for b in range(_R):
                g = go * _R + b
                b2 = (b + 2) % _R
                if b < 2:
                    @pl.when(go > 0)
                    def _():
                        wait_scatter(b2)
                else:
                    wait_scatter(b2)
                if b < 2:
                    wait_it(b2)
                    gather(b2)
                else:
                    @pl.when(go < n_outer - 1)
                    def _():
                        wait_it(b2)
                        gather(b2)
                wait_gather(b)
                compute(b)

                @pl.when(go < n_outer - 1)
                def _():
                    it_copy(g + _R, b)
                scatter(g, b)

        wait_scatter(_R - 2)
        wait_scatter(_R - 1)

    return k, C


def kernel(input_ids, token_type_ids, word_embeddings, token_type_embeddings):
    B, S = input_ids.shape
    V, D = word_embeddings.shape
    T = token_type_embeddings.shape[0]
    N = B * S
    k, C = _build(N, V, D, T)
    ids = input_ids.reshape(N // C, 1, C).astype(jnp.int32)
    tids = token_type_ids.reshape(N // C, 1, C).astype(jnp.int32)
    it = jnp.concatenate([ids, tids], axis=1)
    out = k(it, word_embeddings, token_type_embeddings)
    return out.reshape(B, S, D)
